# Initial kernel scaffold; baseline (speedup 1.0000x reference)
import jax
import jax.numpy as jnp
from jax.experimental import pallas as pl


def kernel(x, edge_index, edge_attr, edge_weight, params):
    raise NotImplementedError("write your pallas kernel here")


# speedup vs baseline: 1.3278x; 1.3278x over previous
import functools
import math

import jax
import jax.numpy as jnp
from jax.experimental import pallas as pl
from jax.experimental.pallas import tpu as pltpu

N = 10000
E = 320000
D = 32
EF = 16
NC = 8
H1 = 8
ETILE = 512

_SQRT2 = math.sqrt(2.0)


def _gelu(x):
    return 0.5 * x * (1.0 + jax.lax.erf(x / _SQRT2))


def _ln_in(x, g, b):
    m = jnp.mean(x, -1, keepdims=True)
    v = jnp.mean((x - m) * (x - m), -1, keepdims=True)
    return (x - m) / jnp.sqrt(v + 1e-5) * g + b


def _lstm_tproj_body(x_ref, wih_f_ref, whhT_f_ref, bb_f_ref,
                     wih_b_ref, whhT_b_ref, bb_b_ref,
                     wp_ref, bp_ref, g_ref, b_ref,
                     out_ref, xih_f, xih_b, hs):
    xcol = x_ref[:, :]
    xih_f[:, :] = jnp.dot(xcol, wih_f_ref[:, :]) + bb_f_ref[:, :]
    xih_b[:, :] = jnp.dot(xcol, wih_b_ref[:, :]) + bb_b_ref[:, :]
    whhT_f = whhT_f_ref[:, :]
    whhT_b = whhT_b_ref[:, :]

    def cell(gpre, c):
        i = jax.nn.sigmoid(gpre[:, 0:D])
        f = jax.nn.sigmoid(gpre[:, D:2 * D])
        gg = jnp.tanh(gpre[:, 2 * D:3 * D])
        o = jax.nn.sigmoid(gpre[:, 3 * D:4 * D])
        c2 = f * c + i * gg
        h2 = o * jnp.tanh(c2)
        return h2, c2

    def step(t, carry):
        hf, cf, hb, cb = carry
        gf = xih_f[pl.ds(t, 1), :] + jnp.dot(hf, whhT_f)
        hf2, cf2 = cell(gf, cf)
        hs[pl.ds(t, 1), 0:D] = hf2
        tb = N - 1 - t
        gb = xih_b[pl.ds(tb, 1), :] + jnp.dot(hb, whhT_b)
        hb2, cb2 = cell(gb, cb)
        hs[pl.ds(tb, 1), D:2 * D] = hb2
        return hf2, cf2, hb2, cb2

    z = jnp.zeros((1, D), jnp.float32)
    jax.lax.fori_loop(0, N, step, (z, z, z, z))

    xt = hs[:, :]
    y = jnp.dot(xt, wp_ref[:, :]) + bp_ref[:, :]
    out_ref[:, :] = _gelu(_ln_in(y, g_ref[:, :], b_ref[:, :]))


def _lstm_tproj(x, pf, pb, tp, tln):
    def prep(p):
        return (p["wih"][:, 0:1].T,
                p["whh"].T,
                (p["bih"] + p["bhh"]).reshape(1, -1))
    wf, hf, bf = prep(pf)
    wb, hb, bb = prep(pb)
    return pl.pallas_call(
        _lstm_tproj_body,
        out_shape=jax.ShapeDtypeStruct((N, D), jnp.float32),
        scratch_shapes=[pltpu.VMEM((N, 4 * D), jnp.float32),
                        pltpu.VMEM((N, 4 * D), jnp.float32),
                        pltpu.VMEM((N, 2 * D), jnp.float32)],
    )(x, wf, hf, bf, wb, hb, bb,
      tp["w"], tp["b"].reshape(1, -1), tln["g"].reshape(1, -1),
      tln["b"].reshape(1, -1))


def _edge_proc_body(ea_ref, ew_ref,
                    w1, b1, g1, bb1, w2, b2, g2, bb2,
                    w3, b3, g3, bb3, w4, b4, out_ref):
    h = jnp.dot(ea_ref[:, :], w1[:, :]) + b1[:, :]
    h = _gelu(_ln_in(h, g1[:, :], bb1[:, :]))
    h = jnp.dot(h, w2[:, :]) + b2[:, :]
    h = _gelu(_ln_in(h, g2[:, :], bb2[:, :]))
    h = jnp.dot(h, w3[:, :]) + b3[:, :]
    h = _gelu(_ln_in(h, g3[:, :], bb3[:, :]))
    h = jnp.dot(h, w4[:, :]) + b4[:, :]
    out_ref[:, :] = jax.nn.sigmoid(h) * ew_ref[:, :]


def _edge_proc(ea, ew, p):
    grid = E // ETILE
    full = lambda shape: pl.BlockSpec(shape, lambda i: (0, 0))
    args = [ea, ew]
    specs = [pl.BlockSpec((ETILE, EF), lambda i: (i, 0)),
             pl.BlockSpec((ETILE, 1), lambda i: (i, 0))]
    for lname, nname in (("l1", "n1"), ("l2", "n2"), ("l3", "n3")):
        w = p[lname]["w"]
        args += [w, p[lname]["b"].reshape(1, -1),
                 p[nname]["g"].reshape(1, -1), p[nname]["b"].reshape(1, -1)]
        specs += [full(w.shape), full((1, w.shape[1])),
                  full((1, w.shape[1])), full((1, w.shape[1]))]
    args += [p["l4"]["w"], p["l4"]["b"].reshape(1, -1)]
    specs += [full(p["l4"]["w"].shape), full((1, 1))]
    return pl.pallas_call(
        _edge_proc_body,
        grid=(grid,),
        in_specs=specs,
        out_specs=pl.BlockSpec((ETILE, 1), lambda i: (i, 0)),
        out_shape=jax.ShapeDtypeStruct((E, 1), jnp.float32),
    )(*args)


def _gat1_dense_body(xt_ref, w_ref, as_ref, ad_ref, xw_ref, s_ref, d_ref):
    xw = jnp.dot(xt_ref[:, :], w_ref[:, :])
    xw_ref[:, :] = xw
    s_ref[:, :] = jnp.dot(xw, as_ref[:, :])
    d_ref[:, :] = jnp.dot(xw, ad_ref[:, :])


def _head_proj_mat(att):
    H, C = att.shape
    rows = jnp.arange(H * C)
    cols = jnp.repeat(jnp.arange(H), C)
    return jnp.zeros((H * C, H), jnp.float32).at[rows, cols].set(att.reshape(-1))


def _gat1_dense(xt, p):
    As = _head_proj_mat(p["att_src"])
    Ad = _head_proj_mat(p["att_dst"])
    return pl.pallas_call(
        _gat1_dense_body,
        out_shape=[jax.ShapeDtypeStruct((N, H1 * D), jnp.float32),
                   jax.ShapeDtypeStruct((N, H1), jnp.float32),
                   jax.ShapeDtypeStruct((N, H1), jnp.float32)],
    )(xt, p["w"], As, Ad)


def _gat2_dense_body(g1raw_ref, bias_ref, g_ref, b_ref, w_ref, as_ref, ad_ref,
                     xw_ref, s_ref, d_ref):
    g1 = _gelu(_ln_in(g1raw_ref[:, :] + bias_ref[:, :], g_ref[:, :], b_ref[:, :]))
    xw = jnp.dot(g1, w_ref[:, :])
    xw_ref[:, :] = xw
    s_ref[:, :] = jnp.dot(xw, as_ref[:, :])
    d_ref[:, :] = jnp.dot(xw, ad_ref[:, :])


def _gat2_dense(g1raw, bias1, gn1, p2):
    return pl.pallas_call(
        _gat2_dense_body,
        out_shape=[jax.ShapeDtypeStruct((N, D), jnp.float32),
                   jax.ShapeDtypeStruct((N, 1), jnp.float32),
                   jax.ShapeDtypeStruct((N, 1), jnp.float32)],
    )(g1raw, bias1.reshape(1, -1), gn1["g"].reshape(1, -1),
      gn1["b"].reshape(1, -1), p2["w"], p2["att_src"].reshape(D, 1),
      p2["att_dst"].reshape(D, 1))


def _finish_body(x_ref, bias_ref, g_ref, b_ref, out_ref):
    out_ref[:, :] = _gelu(_ln_in(x_ref[:, :] + bias_ref[:, :],
                                 g_ref[:, :], b_ref[:, :]))


def _finish(x, bias, ln):
    return pl.pallas_call(
        _finish_body,
        out_shape=jax.ShapeDtypeStruct(x.shape, jnp.float32),
    )(x, bias.reshape(1, -1), ln["g"].reshape(1, -1), ln["b"].reshape(1, -1))


def _sage_body(mean_ref, x_ref, wl_ref, bl_ref, wr_ref, g_ref, b_ref, out_ref):
    y = jnp.dot(mean_ref[:, :], wl_ref[:, :]) + bl_ref[:, :] \
        + jnp.dot(x_ref[:, :], wr_ref[:, :])
    out_ref[:, :] = _gelu(_ln_in(y, g_ref[:, :], b_ref[:, :]))


def _sage_combine(mean, x, p, ln):
    return pl.pallas_call(
        _sage_body,
        out_shape=jax.ShapeDtypeStruct((N, D), jnp.float32),
    )(mean, x, p["lin_l"]["w"], p["lin_l"]["b"].reshape(1, -1),
      p["lin_r"]["w"], ln["g"].reshape(1, -1), ln["b"].reshape(1, -1))


def _edge_cls_body(f_ref, w1, b1, g1, bb1, w2, b2, g2, bb2, w3, b3, out_ref):
    h = jnp.dot(f_ref[:, :], w1[:, :]) + b1[:, :]
    h = _gelu(_ln_in(h, g1[:, :], bb1[:, :]))
    h = jnp.dot(h, w2[:, :]) + b2[:, :]
    h = _gelu(_ln_in(h, g2[:, :], bb2[:, :]))
    out_ref[:, :] = jnp.dot(h, w3[:, :]) + b3[:, :]


def _edge_cls(feat, p):
    grid = E // ETILE
    F = feat.shape[1]
    full = lambda shape: pl.BlockSpec(shape, lambda i: (0, 0))
    args = [feat]
    specs = [pl.BlockSpec((ETILE, F), lambda i: (i, 0))]
    for lname, nname in (("l1", "n1"), ("l2", "n2")):
        w = p[lname]["w"]
        args += [w, p[lname]["b"].reshape(1, -1),
                 p[nname]["g"].reshape(1, -1), p[nname]["b"].reshape(1, -1)]
        specs += [full(w.shape), full((1, w.shape[1])),
                  full((1, w.shape[1])), full((1, w.shape[1]))]
    args += [p["l3"]["w"], p["l3"]["b"].reshape(1, -1)]
    specs += [full(p["l3"]["w"].shape), full((1, NC))]
    return pl.pallas_call(
        _edge_cls_body,
        grid=(grid,),
        in_specs=specs,
        out_specs=pl.BlockSpec((ETILE, NC), lambda i: (i, 0)),
        out_shape=jax.ShapeDtypeStruct((E, NC), jnp.float32),
    )(*args)


def kernel(x, edge_index, edge_attr, edge_weight, params):
    src = edge_index[0]
    dst = edge_index[1]
    loop = jnp.arange(N, dtype=src.dtype)
    src2 = jnp.concatenate([src, loop])
    dst2 = jnp.concatenate([dst, loop])

    xt = _lstm_tproj(x, params["lstm_f"], params["lstm_b"],
                     params["tproj"], params["tproj_ln"])

    eap = _edge_proc(edge_attr, edge_weight, params["ep"])[:, 0]
    ea2 = jnp.concatenate([eap, jnp.broadcast_to(jnp.mean(eap), (N,))])

    p1 = params["gat1"]
    xw1, a1s, a1d = _gat1_dense(xt, p1)
    c1 = jnp.sum(p1["w_edge"].reshape(H1, D) * p1["att_edge"], -1)
    alpha = a1s[src2] + a1d[dst2] + ea2[:, None] * c1[None, :]
    alpha = jax.nn.leaky_relu(alpha, negative_slope=0.2)
    amax = jax.ops.segment_max(alpha, dst2, num_segments=N)
    amax = jnp.where(jnp.isfinite(amax), amax, 0.0)
    ex = jnp.exp(alpha - amax[dst2])
    den = jax.ops.segment_sum(ex, dst2, num_segments=N)
    attn = ex / (den[dst2] + 1e-16)
    msg = xw1[src2].reshape(-1, H1, D) * attn[:, :, None]
    g1raw = jax.ops.segment_sum(msg, dst2, num_segments=N).reshape(N, H1 * D)

    p2 = params["gat2"]
    xw2, a2s, a2d = _gat2_dense(g1raw, p1["bias"], params["gn1"], p2)
    c2 = jnp.sum(p2["w_edge"].reshape(1, D) * p2["att_edge"], -1)
    alpha2 = a2s[src2, 0] + a2d[dst2, 0] + ea2 * c2[0]
    alpha2 = jax.nn.leaky_relu(alpha2, negative_slope=0.2)
    amax2 = jax.ops.segment_max(alpha2, dst2, num_segments=N)
    amax2 = jnp.where(jnp.isfinite(amax2), amax2, 0.0)
    ex2 = jnp.exp(alpha2 - amax2[dst2])
    den2 = jax.ops.segment_sum(ex2, dst2, num_segments=N)
    attn2 = ex2 / (den2[dst2] + 1e-16)
    g2raw = jax.ops.segment_sum(xw2[src2] * attn2[:, None], dst2, num_segments=N)
    g2 = _finish(g2raw, p2["bias"], params["gn2"])

    cnt = jax.ops.segment_sum(jnp.ones((E,), jnp.float32), dst, num_segments=N)
    cnt = jnp.clip(cnt, 1.0, None)[:, None]

    w1e = params["sage1"]["scale"] * eap ** 2
    s1sum = jax.ops.segment_sum(g2[src] * w1e[:, None], dst, num_segments=N)
    s1 = _sage_combine(s1sum / cnt, g2, params["sage1"], params["sn1"])

    w2e = params["sage2"]["scale"] * eap ** 2
    s2sum = jax.ops.segment_sum(s1[src] * w2e[:, None], dst, num_segments=N)
    s2 = _sage_combine(s2sum / cnt, s1, params["sage2"], params["sn2"])

    g2s, g2d = g2[src], g2[dst]
    s2s, s2d = s2[src], s2[dst]
    feat = jnp.concatenate([s2s, s2d, eap[:, None]], -1)
    scores = _edge_cls(feat, params["ec"])
    gat_emb = jnp.concatenate([g2s, g2d], -1)
    sage_emb = jnp.concatenate([s2s, s2d], -1)
    return scores, gat_emb, sage_emb


# speedup vs baseline: 1.3398x; 1.0090x over previous
import functools
import math

import jax
import jax.numpy as jnp
from jax.experimental import pallas as pl
from jax.experimental.pallas import tpu as pltpu

N = 10000
E = 320000
D = 32
EF = 16
NC = 8
H1 = 8
ETILE = 512

_SQRT2 = math.sqrt(2.0)


def _gelu(x):
    return 0.5 * x * (1.0 + jax.lax.erf(x / _SQRT2))


def _ln_in(x, g, b):
    m = jnp.mean(x, -1, keepdims=True)
    v = jnp.mean((x - m) * (x - m), -1, keepdims=True)
    return (x - m) / jnp.sqrt(v + 1e-5) * g + b


def _lstm_tproj_body(x_ref, wih_f_ref, whhT_f_ref, bb_f_ref,
                     wih_b_ref, whhT_b_ref, bb_b_ref,
                     wp_ref, bp_ref, g_ref, b_ref,
                     out_ref, xih_f, xih_b, hs):
    xcol = x_ref[:, :]
    xih_f[:, :] = jnp.dot(xcol, wih_f_ref[:, :]) + bb_f_ref[:, :]
    xih_b[:, :] = jnp.dot(xcol, wih_b_ref[:, :]) + bb_b_ref[:, :]
    whhT_f = whhT_f_ref[:, :]
    whhT_b = whhT_b_ref[:, :]

    def cell(gpre, c):
        i = jax.nn.sigmoid(gpre[:, 0:D])
        f = jax.nn.sigmoid(gpre[:, D:2 * D])
        gg = jnp.tanh(gpre[:, 2 * D:3 * D])
        o = jax.nn.sigmoid(gpre[:, 3 * D:4 * D])
        c2 = f * c + i * gg
        h2 = o * jnp.tanh(c2)
        return h2, c2

    def step(t, carry):
        hf, cf, hb, cb = carry
        gf = xih_f[pl.ds(t, 1), :] + jnp.dot(hf, whhT_f)
        hf2, cf2 = cell(gf, cf)
        hs[pl.ds(t, 1), 0:D] = hf2
        tb = N - 1 - t
        gb = xih_b[pl.ds(tb, 1), :] + jnp.dot(hb, whhT_b)
        hb2, cb2 = cell(gb, cb)
        hs[pl.ds(tb, 1), D:2 * D] = hb2
        return hf2, cf2, hb2, cb2

    z = jnp.zeros((1, D), jnp.float32)
    jax.lax.fori_loop(0, N, step, (z, z, z, z), unroll=8)

    xt = hs[:, :]
    y = jnp.dot(xt, wp_ref[:, :]) + bp_ref[:, :]
    out_ref[:, :] = _gelu(_ln_in(y, g_ref[:, :], b_ref[:, :]))


def _lstm_tproj(x, pf, pb, tp, tln):
    def prep(p):
        return (p["wih"][:, 0:1].T,
                p["whh"].T,
                (p["bih"] + p["bhh"]).reshape(1, -1))
    wf, hf, bf = prep(pf)
    wb, hb, bb = prep(pb)
    return pl.pallas_call(
        _lstm_tproj_body,
        out_shape=jax.ShapeDtypeStruct((N, D), jnp.float32),
        scratch_shapes=[pltpu.VMEM((N, 4 * D), jnp.float32),
                        pltpu.VMEM((N, 4 * D), jnp.float32),
                        pltpu.VMEM((N, 2 * D), jnp.float32)],
    )(x, wf, hf, bf, wb, hb, bb,
      tp["w"], tp["b"].reshape(1, -1), tln["g"].reshape(1, -1),
      tln["b"].reshape(1, -1))


def _edge_proc_body(ea_ref, ew_ref,
                    w1, b1, g1, bb1, w2, b2, g2, bb2,
                    w3, b3, g3, bb3, w4, b4, out_ref):
    h = jnp.dot(ea_ref[:, :], w1[:, :]) + b1[:, :]
    h = _gelu(_ln_in(h, g1[:, :], bb1[:, :]))
    h = jnp.dot(h, w2[:, :]) + b2[:, :]
    h = _gelu(_ln_in(h, g2[:, :], bb2[:, :]))
    h = jnp.dot(h, w3[:, :]) + b3[:, :]
    h = _gelu(_ln_in(h, g3[:, :], bb3[:, :]))
    h = jnp.dot(h, w4[:, :]) + b4[:, :]
    out_ref[:, :] = jax.nn.sigmoid(h) * ew_ref[:, :]


def _edge_proc(ea, ew, p):
    grid = E // ETILE
    full = lambda shape: pl.BlockSpec(shape, lambda i: (0, 0))
    args = [ea, ew]
    specs = [pl.BlockSpec((ETILE, EF), lambda i: (i, 0)),
             pl.BlockSpec((ETILE, 1), lambda i: (i, 0))]
    for lname, nname in (("l1", "n1"), ("l2", "n2"), ("l3", "n3")):
        w = p[lname]["w"]
        args += [w, p[lname]["b"].reshape(1, -1),
                 p[nname]["g"].reshape(1, -1), p[nname]["b"].reshape(1, -1)]
        specs += [full(w.shape), full((1, w.shape[1])),
                  full((1, w.shape[1])), full((1, w.shape[1]))]
    args += [p["l4"]["w"], p["l4"]["b"].reshape(1, -1)]
    specs += [full(p["l4"]["w"].shape), full((1, 1))]
    return pl.pallas_call(
        _edge_proc_body,
        grid=(grid,),
        in_specs=specs,
        out_specs=pl.BlockSpec((ETILE, 1), lambda i: (i, 0)),
        out_shape=jax.ShapeDtypeStruct((E, 1), jnp.float32),
    )(*args)


def _gat1_dense_body(xt_ref, w_ref, as_ref, ad_ref, xw_ref, s_ref, d_ref):
    xw = jnp.dot(xt_ref[:, :], w_ref[:, :])
    xw_ref[:, :] = xw
    s_ref[:, :] = jnp.dot(xw, as_ref[:, :])
    d_ref[:, :] = jnp.dot(xw, ad_ref[:, :])


def _head_proj_mat(att):
    H, C = att.shape
    rows = jnp.arange(H * C)
    cols = jnp.repeat(jnp.arange(H), C)
    return jnp.zeros((H * C, H), jnp.float32).at[rows, cols].set(att.reshape(-1))


def _gat1_dense(xt, p):
    As = _head_proj_mat(p["att_src"])
    Ad = _head_proj_mat(p["att_dst"])
    return pl.pallas_call(
        _gat1_dense_body,
        out_shape=[jax.ShapeDtypeStruct((N, H1 * D), jnp.float32),
                   jax.ShapeDtypeStruct((N, H1), jnp.float32),
                   jax.ShapeDtypeStruct((N, H1), jnp.float32)],
    )(xt, p["w"], As, Ad)


def _gat2_dense_body(g1raw_ref, bias_ref, g_ref, b_ref, w_ref, as_ref, ad_ref,
                     xw_ref, s_ref, d_ref):
    g1 = _gelu(_ln_in(g1raw_ref[:, :] + bias_ref[:, :], g_ref[:, :], b_ref[:, :]))
    xw = jnp.dot(g1, w_ref[:, :])
    xw_ref[:, :] = xw
    s_ref[:, :] = jnp.dot(xw, as_ref[:, :])
    d_ref[:, :] = jnp.dot(xw, ad_ref[:, :])


def _gat2_dense(g1raw, bias1, gn1, p2):
    return pl.pallas_call(
        _gat2_dense_body,
        out_shape=[jax.ShapeDtypeStruct((N, D), jnp.float32),
                   jax.ShapeDtypeStruct((N, 1), jnp.float32),
                   jax.ShapeDtypeStruct((N, 1), jnp.float32)],
    )(g1raw, bias1.reshape(1, -1), gn1["g"].reshape(1, -1),
      gn1["b"].reshape(1, -1), p2["w"], p2["att_src"].reshape(D, 1),
      p2["att_dst"].reshape(D, 1))


def _finish_body(x_ref, bias_ref, g_ref, b_ref, out_ref):
    out_ref[:, :] = _gelu(_ln_in(x_ref[:, :] + bias_ref[:, :],
                                 g_ref[:, :], b_ref[:, :]))


def _finish(x, bias, ln):
    return pl.pallas_call(
        _finish_body,
        out_shape=jax.ShapeDtypeStruct(x.shape, jnp.float32),
    )(x, bias.reshape(1, -1), ln["g"].reshape(1, -1), ln["b"].reshape(1, -1))


def _sage_body(mean_ref, x_ref, wl_ref, bl_ref, wr_ref, g_ref, b_ref, out_ref):
    y = jnp.dot(mean_ref[:, :], wl_ref[:, :]) + bl_ref[:, :] \
        + jnp.dot(x_ref[:, :], wr_ref[:, :])
    out_ref[:, :] = _gelu(_ln_in(y, g_ref[:, :], b_ref[:, :]))


def _sage_combine(mean, x, p, ln):
    return pl.pallas_call(
        _sage_body,
        out_shape=jax.ShapeDtypeStruct((N, D), jnp.float32),
    )(mean, x, p["lin_l"]["w"], p["lin_l"]["b"].reshape(1, -1),
      p["lin_r"]["w"], ln["g"].reshape(1, -1), ln["b"].reshape(1, -1))


def _edge_cls_body(f_ref, w1, b1, g1, bb1, w2, b2, g2, bb2, w3, b3, out_ref):
    h = jnp.dot(f_ref[:, :], w1[:, :]) + b1[:, :]
    h = _gelu(_ln_in(h, g1[:, :], bb1[:, :]))
    h = jnp.dot(h, w2[:, :]) + b2[:, :]
    h = _gelu(_ln_in(h, g2[:, :], bb2[:, :]))
    out_ref[:, :] = jnp.dot(h, w3[:, :]) + b3[:, :]


def _edge_cls(feat, p):
    grid = E // ETILE
    F = feat.shape[1]
    full = lambda shape: pl.BlockSpec(shape, lambda i: (0, 0))
    args = [feat]
    specs = [pl.BlockSpec((ETILE, F), lambda i: (i, 0))]
    for lname, nname in (("l1", "n1"), ("l2", "n2")):
        w = p[lname]["w"]
        args += [w, p[lname]["b"].reshape(1, -1),
                 p[nname]["g"].reshape(1, -1), p[nname]["b"].reshape(1, -1)]
        specs += [full(w.shape), full((1, w.shape[1])),
                  full((1, w.shape[1])), full((1, w.shape[1]))]
    args += [p["l3"]["w"], p["l3"]["b"].reshape(1, -1)]
    specs += [full(p["l3"]["w"].shape), full((1, NC))]
    return pl.pallas_call(
        _edge_cls_body,
        grid=(grid,),
        in_specs=specs,
        out_specs=pl.BlockSpec((ETILE, NC), lambda i: (i, 0)),
        out_shape=jax.ShapeDtypeStruct((E, NC), jnp.float32),
    )(*args)


def kernel(x, edge_index, edge_attr, edge_weight, params):
    src = edge_index[0]
    dst = edge_index[1]
    loop = jnp.arange(N, dtype=src.dtype)
    src2 = jnp.concatenate([src, loop])
    dst2 = jnp.concatenate([dst, loop])

    xt = _lstm_tproj(x, params["lstm_f"], params["lstm_b"],
                     params["tproj"], params["tproj_ln"])

    eap = _edge_proc(edge_attr, edge_weight, params["ep"])[:, 0]
    ea2 = jnp.concatenate([eap, jnp.broadcast_to(jnp.mean(eap), (N,))])

    p1 = params["gat1"]
    xw1, a1s, a1d = _gat1_dense(xt, p1)
    c1 = jnp.sum(p1["w_edge"].reshape(H1, D) * p1["att_edge"], -1)
    alpha = a1s[src2] + a1d[dst2] + ea2[:, None] * c1[None, :]
    alpha = jax.nn.leaky_relu(alpha, negative_slope=0.2)
    amax = jax.ops.segment_max(alpha, dst2, num_segments=N)
    amax = jnp.where(jnp.isfinite(amax), amax, 0.0)
    ex = jnp.exp(alpha - amax[dst2])
    den = jax.ops.segment_sum(ex, dst2, num_segments=N)
    attn = ex / (den[dst2] + 1e-16)
    msg = xw1[src2].reshape(-1, H1, D) * attn[:, :, None]
    g1raw = jax.ops.segment_sum(msg, dst2, num_segments=N).reshape(N, H1 * D)

    p2 = params["gat2"]
    xw2, a2s, a2d = _gat2_dense(g1raw, p1["bias"], params["gn1"], p2)
    c2 = jnp.sum(p2["w_edge"].reshape(1, D) * p2["att_edge"], -1)
    alpha2 = a2s[src2, 0] + a2d[dst2, 0] + ea2 * c2[0]
    alpha2 = jax.nn.leaky_relu(alpha2, negative_slope=0.2)
    amax2 = jax.ops.segment_max(alpha2, dst2, num_segments=N)
    amax2 = jnp.where(jnp.isfinite(amax2), amax2, 0.0)
    ex2 = jnp.exp(alpha2 - amax2[dst2])
    den2 = jax.ops.segment_sum(ex2, dst2, num_segments=N)
    attn2 = ex2 / (den2[dst2] + 1e-16)
    g2raw = jax.ops.segment_sum(xw2[src2] * attn2[:, None], dst2, num_segments=N)
    g2 = _finish(g2raw, p2["bias"], params["gn2"])

    cnt = jax.ops.segment_sum(jnp.ones((E,), jnp.float32), dst, num_segments=N)
    cnt = jnp.clip(cnt, 1.0, None)[:, None]

    w1e = params["sage1"]["scale"] * eap ** 2
    s1sum = jax.ops.segment_sum(g2[src] * w1e[:, None], dst, num_segments=N)
    s1 = _sage_combine(s1sum / cnt, g2, params["sage1"], params["sn1"])

    w2e = params["sage2"]["scale"] * eap ** 2
    s2sum = jax.ops.segment_sum(s1[src] * w2e[:, None], dst, num_segments=N)
    s2 = _sage_combine(s2sum / cnt, s1, params["sage2"], params["sn2"])

    g2s, g2d = g2[src], g2[dst]
    s2s, s2d = s2[src], s2[dst]
    feat = jnp.concatenate([s2s, s2d, eap[:, None]], -1)
    scores = _edge_cls(feat, params["ec"])
    gat_emb = jnp.concatenate([g2s, g2d], -1)
    sage_emb = jnp.concatenate([s2s, s2d], -1)
    return scores, gat_emb, sage_emb


# speedup vs baseline: 2.6148x; 1.9516x over previous
import functools
import math

import jax
import jax.numpy as jnp
from jax.experimental import pallas as pl
from jax.experimental.pallas import tpu as pltpu

N = 10000
E = 320000
D = 32
EF = 16
NC = 8
H1 = 8
ETILE = 512

_SQRT2 = math.sqrt(2.0)


def _gelu(x):
    return 0.5 * x * (1.0 + jax.lax.erf(x / _SQRT2))


def _ln_in(x, g, b):
    m = jnp.mean(x, -1, keepdims=True)
    v = jnp.mean((x - m) * (x - m), -1, keepdims=True)
    return (x - m) / jnp.sqrt(v + 1e-5) * g + b


def _lstm_tproj_body(x_ref, wih_f_ref, whhT_f_ref, bb_f_ref,
                     wih_b_ref, whhT_b_ref, bb_b_ref,
                     wp_ref, bp_ref, g_ref, b_ref,
                     out_ref, xih_f, xih_b, hs):
    xcol = x_ref[:, :]
    xih_f[:, :] = jnp.dot(xcol, wih_f_ref[:, :]) + bb_f_ref[:, :]
    xih_b[:, :] = jnp.dot(xcol, wih_b_ref[:, :]) + bb_b_ref[:, :]
    whhT_f = whhT_f_ref[:, :]
    whhT_b = whhT_b_ref[:, :]

    def cell(gpre, c):
        i = jax.nn.sigmoid(gpre[:, 0:D])
        f = jax.nn.sigmoid(gpre[:, D:2 * D])
        gg = jnp.tanh(gpre[:, 2 * D:3 * D])
        o = jax.nn.sigmoid(gpre[:, 3 * D:4 * D])
        c2 = f * c + i * gg
        h2 = o * jnp.tanh(c2)
        return h2, c2

    def step(t, carry):
        hf, cf, hb, cb = carry
        gf = xih_f[pl.ds(t, 1), :] + jnp.dot(hf, whhT_f)
        hf2, cf2 = cell(gf, cf)
        hs[pl.ds(t, 1), 0:D] = hf2
        tb = N - 1 - t
        gb = xih_b[pl.ds(tb, 1), :] + jnp.dot(hb, whhT_b)
        hb2, cb2 = cell(gb, cb)
        hs[pl.ds(tb, 1), D:2 * D] = hb2
        return hf2, cf2, hb2, cb2

    z = jnp.zeros((1, D), jnp.float32)
    jax.lax.fori_loop(0, N, step, (z, z, z, z), unroll=8)

    xt = hs[:, :]
    y = jnp.dot(xt, wp_ref[:, :]) + bp_ref[:, :]
    out_ref[:, :] = _gelu(_ln_in(y, g_ref[:, :], b_ref[:, :]))


def _lstm_tproj(x, pf, pb, tp, tln):
    def prep(p):
        return (p["wih"][:, 0:1].T,
                p["whh"].T,
                (p["bih"] + p["bhh"]).reshape(1, -1))
    wf, hf, bf = prep(pf)
    wb, hb, bb = prep(pb)
    return pl.pallas_call(
        _lstm_tproj_body,
        out_shape=jax.ShapeDtypeStruct((N, D), jnp.float32),
        scratch_shapes=[pltpu.VMEM((N, 4 * D), jnp.float32),
                        pltpu.VMEM((N, 4 * D), jnp.float32),
                        pltpu.VMEM((N, 2 * D), jnp.float32)],
    )(x, wf, hf, bf, wb, hb, bb,
      tp["w"], tp["b"].reshape(1, -1), tln["g"].reshape(1, -1),
      tln["b"].reshape(1, -1))


def _edge_proc_body(ea_ref, ew_ref,
                    w1, b1, g1, bb1, w2, b2, g2, bb2,
                    w3, b3, g3, bb3, w4, b4, out_ref):
    h = jnp.dot(ea_ref[:, :], w1[:, :]) + b1[:, :]
    h = _gelu(_ln_in(h, g1[:, :], bb1[:, :]))
    h = jnp.dot(h, w2[:, :]) + b2[:, :]
    h = _gelu(_ln_in(h, g2[:, :], bb2[:, :]))
    h = jnp.dot(h, w3[:, :]) + b3[:, :]
    h = _gelu(_ln_in(h, g3[:, :], bb3[:, :]))
    h = jnp.dot(h, w4[:, :]) + b4[:, :]
    out_ref[:, :] = jax.nn.sigmoid(h) * ew_ref[:, :]


def _edge_proc(ea, ew, p):
    grid = E // ETILE
    full = lambda shape: pl.BlockSpec(shape, lambda i: (0, 0))
    args = [ea, ew]
    specs = [pl.BlockSpec((ETILE, EF), lambda i: (i, 0)),
             pl.BlockSpec((ETILE, 1), lambda i: (i, 0))]
    for lname, nname in (("l1", "n1"), ("l2", "n2"), ("l3", "n3")):
        w = p[lname]["w"]
        args += [w, p[lname]["b"].reshape(1, -1),
                 p[nname]["g"].reshape(1, -1), p[nname]["b"].reshape(1, -1)]
        specs += [full(w.shape), full((1, w.shape[1])),
                  full((1, w.shape[1])), full((1, w.shape[1]))]
    args += [p["l4"]["w"], p["l4"]["b"].reshape(1, -1)]
    specs += [full(p["l4"]["w"].shape), full((1, 1))]
    return pl.pallas_call(
        _edge_proc_body,
        grid=(grid,),
        in_specs=specs,
        out_specs=pl.BlockSpec((ETILE, 1), lambda i: (i, 0)),
        out_shape=jax.ShapeDtypeStruct((E, 1), jnp.float32),
    )(*args)


def _gat1_dense_body(xt_ref, w_ref, as_ref, ad_ref, xw_ref, s_ref, d_ref):
    xw = jnp.dot(xt_ref[:, :], w_ref[:, :])
    xw_ref[:, :] = xw
    s_ref[:, :] = jnp.dot(xw, as_ref[:, :])
    d_ref[:, :] = jnp.dot(xw, ad_ref[:, :])


def _head_proj_mat(att):
    H, C = att.shape
    rows = jnp.arange(H * C)
    cols = jnp.repeat(jnp.arange(H), C)
    return jnp.zeros((H * C, H), jnp.float32).at[rows, cols].set(att.reshape(-1))


def _gat1_dense(xt, p):
    As = _head_proj_mat(p["att_src"])
    Ad = _head_proj_mat(p["att_dst"])
    return pl.pallas_call(
        _gat1_dense_body,
        out_shape=[jax.ShapeDtypeStruct((N, H1 * D), jnp.float32),
                   jax.ShapeDtypeStruct((N, H1), jnp.float32),
                   jax.ShapeDtypeStruct((N, H1), jnp.float32)],
    )(xt, p["w"], As, Ad)


def _gat2_dense_body(g1raw_ref, bias_ref, g_ref, b_ref, w_ref, as_ref, ad_ref,
                     xw_ref, s_ref, d_ref):
    g1 = _gelu(_ln_in(g1raw_ref[:, :] + bias_ref[:, :], g_ref[:, :], b_ref[:, :]))
    xw = jnp.dot(g1, w_ref[:, :])
    xw_ref[:, :] = xw
    s_ref[:, :] = jnp.dot(xw, as_ref[:, :])
    d_ref[:, :] = jnp.dot(xw, ad_ref[:, :])


def _gat2_dense(g1raw, bias1, gn1, p2):
    return pl.pallas_call(
        _gat2_dense_body,
        out_shape=[jax.ShapeDtypeStruct((N, D), jnp.float32),
                   jax.ShapeDtypeStruct((N, 1), jnp.float32),
                   jax.ShapeDtypeStruct((N, 1), jnp.float32)],
    )(g1raw, bias1.reshape(1, -1), gn1["g"].reshape(1, -1),
      gn1["b"].reshape(1, -1), p2["w"], p2["att_src"].reshape(D, 1),
      p2["att_dst"].reshape(D, 1))


def _finish_body(x_ref, bias_ref, g_ref, b_ref, out_ref):
    out_ref[:, :] = _gelu(_ln_in(x_ref[:, :] + bias_ref[:, :],
                                 g_ref[:, :], b_ref[:, :]))


def _finish(x, bias, ln):
    return pl.pallas_call(
        _finish_body,
        out_shape=jax.ShapeDtypeStruct(x.shape, jnp.float32),
    )(x, bias.reshape(1, -1), ln["g"].reshape(1, -1), ln["b"].reshape(1, -1))


def _sage_body(mean_ref, x_ref, wl_ref, bl_ref, wr_ref, g_ref, b_ref, out_ref):
    y = jnp.dot(mean_ref[:, :], wl_ref[:, :]) + bl_ref[:, :] \
        + jnp.dot(x_ref[:, :], wr_ref[:, :])
    out_ref[:, :] = _gelu(_ln_in(y, g_ref[:, :], b_ref[:, :]))


def _sage_combine(mean, x, p, ln):
    return pl.pallas_call(
        _sage_body,
        out_shape=jax.ShapeDtypeStruct((N, D), jnp.float32),
    )(mean, x, p["lin_l"]["w"], p["lin_l"]["b"].reshape(1, -1),
      p["lin_r"]["w"], ln["g"].reshape(1, -1), ln["b"].reshape(1, -1))


def _edge_cls_body(f_ref, w1, b1, g1, bb1, w2, b2, g2, bb2, w3, b3, out_ref):
    h = jnp.dot(f_ref[:, :], w1[:, :]) + b1[:, :]
    h = _gelu(_ln_in(h, g1[:, :], bb1[:, :]))
    h = jnp.dot(h, w2[:, :]) + b2[:, :]
    h = _gelu(_ln_in(h, g2[:, :], bb2[:, :]))
    out_ref[:, :] = jnp.dot(h, w3[:, :]) + b3[:, :]


def _edge_cls(feat, p):
    grid = E // ETILE
    F = feat.shape[1]
    full = lambda shape: pl.BlockSpec(shape, lambda i: (0, 0))
    args = [feat]
    specs = [pl.BlockSpec((ETILE, F), lambda i: (i, 0))]
    for lname, nname in (("l1", "n1"), ("l2", "n2")):
        w = p[lname]["w"]
        args += [w, p[lname]["b"].reshape(1, -1),
                 p[nname]["g"].reshape(1, -1), p[nname]["b"].reshape(1, -1)]
        specs += [full(w.shape), full((1, w.shape[1])),
                  full((1, w.shape[1])), full((1, w.shape[1]))]
    args += [p["l3"]["w"], p["l3"]["b"].reshape(1, -1)]
    specs += [full(p["l3"]["w"].shape), full((1, NC))]
    return pl.pallas_call(
        _edge_cls_body,
        grid=(grid,),
        in_specs=specs,
        out_specs=pl.BlockSpec((ETILE, NC), lambda i: (i, 0)),
        out_shape=jax.ShapeDtypeStruct((E, NC), jnp.float32),
    )(*args)


def kernel(x, edge_index, edge_attr, edge_weight, params):
    src = edge_index[0]
    dst = edge_index[1]
    loop = jnp.arange(N, dtype=src.dtype)
    src2 = jnp.concatenate([src, loop])
    dst2 = jnp.concatenate([dst, loop])

    xt = _lstm_tproj(x, params["lstm_f"], params["lstm_b"],
                     params["tproj"], params["tproj_ln"])

    eap = _edge_proc(edge_attr, edge_weight, params["ep"])[:, 0]
    ea2 = jnp.concatenate([eap, jnp.broadcast_to(jnp.mean(eap), (N,))])

    p1 = params["gat1"]
    xw1, a1s, a1d = _gat1_dense(xt, p1)
    c1 = jnp.sum(p1["w_edge"].reshape(H1, D) * p1["att_edge"], -1)
    alpha = a1s[src2] + a1d[dst2] + ea2[:, None] * c1[None, :]
    alpha = jax.nn.leaky_relu(alpha, negative_slope=0.2)
    amax = jax.ops.segment_max(alpha, dst2, num_segments=N)
    amax = jnp.where(jnp.isfinite(amax), amax, 0.0)
    ex = jnp.exp(alpha - amax[dst2])
    den = jax.ops.segment_sum(ex, dst2, num_segments=N)
    attn = ex / (den[dst2] + 1e-16)
    msg = (xw1[src2].reshape(-1, H1, D) * attn[:, :, None]).reshape(-1, H1 * D)
    g1raw = jax.ops.segment_sum(msg, dst2, num_segments=N)

    p2 = params["gat2"]
    xw2, a2s, a2d = _gat2_dense(g1raw, p1["bias"], params["gn1"], p2)
    c2 = jnp.sum(p2["w_edge"].reshape(1, D) * p2["att_edge"], -1)
    alpha2 = a2s[src2, 0] + a2d[dst2, 0] + ea2 * c2[0]
    alpha2 = jax.nn.leaky_relu(alpha2, negative_slope=0.2)
    amax2 = jax.ops.segment_max(alpha2, dst2, num_segments=N)
    amax2 = jnp.where(jnp.isfinite(amax2), amax2, 0.0)
    ex2 = jnp.exp(alpha2 - amax2[dst2])
    den2 = jax.ops.segment_sum(ex2, dst2, num_segments=N)
    attn2 = ex2 / (den2[dst2] + 1e-16)
    g2raw = jax.ops.segment_sum(xw2[src2] * attn2[:, None], dst2, num_segments=N)
    g2 = _finish(g2raw, p2["bias"], params["gn2"])

    cnt = jax.ops.segment_sum(jnp.ones((E,), jnp.float32), dst, num_segments=N)
    cnt = jnp.clip(cnt, 1.0, None)[:, None]

    w1e = params["sage1"]["scale"] * eap ** 2
    s1sum = jax.ops.segment_sum(g2[src] * w1e[:, None], dst, num_segments=N)
    s1 = _sage_combine(s1sum / cnt, g2, params["sage1"], params["sn1"])

    w2e = params["sage2"]["scale"] * eap ** 2
    s2sum = jax.ops.segment_sum(s1[src] * w2e[:, None], dst, num_segments=N)
    s2 = _sage_combine(s2sum / cnt, s1, params["sage2"], params["sn2"])

    g2s, g2d = g2[src], g2[dst]
    s2s, s2d = s2[src], s2[dst]
    feat = jnp.concatenate([s2s, s2d, eap[:, None]], -1)
    scores = _edge_cls(feat, params["ec"])
    gat_emb = jnp.concatenate([g2s, g2d], -1)
    sage_emb = jnp.concatenate([s2s, s2d], -1)
    return scores, gat_emb, sage_emb
